# baseline (speedup 1.0000x reference)
import jax
import jax.numpy as jnp
from jax.experimental import pallas as pl

N = 32768
D = 64
FF = 48

BN = 8192


def _moe_block(x_ref, sw1_ref, sb1_ref, sw2_ref, sb2_ref,
               rw1_ref, rb1_ref, rw2_ref, rb2_ref, gw_ref, gb_ref,
               out_ref):
    x = x_ref[...]

    def expert(w1, b1, w2, b2):
        h = jnp.dot(x, w1, preferred_element_type=jnp.float32) + b1
        a = h[:, :FF]
        b = h[:, FF:]
        act = (a * jax.nn.sigmoid(a)) * b
        return jnp.dot(act, w2, preferred_element_type=jnp.float32) + b2

    shared = expert(sw1_ref[...], sb1_ref[...], sw2_ref[...], sb2_ref[...])
    o0 = expert(rw1_ref[0], rb1_ref[0], rw2_ref[0], rb2_ref[0])
    o1 = expert(rw1_ref[1], rb1_ref[1], rw2_ref[1], rb2_ref[1])

    logits = jnp.dot(x, gw_ref[...], preferred_element_type=jnp.float32) + gb_ref[...]
    l0 = logits[:, 0:1]
    l1 = logits[:, 1:2]
    pick1 = l1 > l0
    w = jax.nn.sigmoid(jnp.abs(l1 - l0))
    routed = jnp.where(pick1, o1, o0) * w
    out_ref[...] = shared + routed


@jax.jit
def kernel(x, sw1, sb1, sw2, sb2, rw1, rb1, rw2, rb2, gw, gb):
    grid = (N // BN,)
    full = lambda *s: pl.BlockSpec(s, lambda i: (0,) * len(s))
    return pl.pallas_call(
        _moe_block,
        grid=grid,
        in_specs=[
            pl.BlockSpec((BN, D), lambda i: (i, 0)),
            full(D, 2 * FF), full(2 * FF), full(FF, D), full(D),
            full(2, D, 2 * FF), full(2, 2 * FF), full(2, FF, D), full(2, D),
            full(D, 2), full(2),
        ],
        out_specs=pl.BlockSpec((BN, D), lambda i: (i, 0)),
        out_shape=jax.ShapeDtypeStruct((N, D), jnp.float32),
    )(x, sw1, sb1, sw2, sb2, rw1, rb1, rw2, rb2, gw, gb)


# speedup vs baseline: 1.0026x; 1.0026x over previous
import jax
import jax.numpy as jnp
from jax.experimental import pallas as pl

N = 32768
D = 64
FF = 48

BN = 4096


def _moe_block(x_ref, sw1_ref, sb1_ref, sw2_ref, sb2_ref,
               rw1_ref, rb1_ref, rw2_ref, rb2_ref, gw_ref, gb_ref,
               out_ref):
    xf = x_ref[...]
    x = xf.astype(jnp.bfloat16)

    def expert(w1, b1, w2, b2):
        h = jnp.dot(x, w1.astype(jnp.bfloat16),
                    preferred_element_type=jnp.float32) + b1
        a = h[:, :FF]
        b = h[:, FF:]
        act = ((a * jax.nn.sigmoid(a)) * b).astype(jnp.bfloat16)
        return jnp.dot(act, w2.astype(jnp.bfloat16),
                       preferred_element_type=jnp.float32) + b2

    shared = expert(sw1_ref[...], sb1_ref[...], sw2_ref[...], sb2_ref[...])
    o0 = expert(rw1_ref[0], rb1_ref[0], rw2_ref[0], rb2_ref[0])
    o1 = expert(rw1_ref[1], rb1_ref[1], rw2_ref[1], rb2_ref[1])

    logits = jnp.dot(xf, gw_ref[...], preferred_element_type=jnp.float32) + gb_ref[...]
    l0 = logits[:, 0:1]
    l1 = logits[:, 1:2]
    pick1 = l1 > l0
    w = jax.nn.sigmoid(jnp.abs(l1 - l0))
    routed = jnp.where(pick1, o1, o0) * w
    out_ref[...] = shared + routed


@jax.jit
def kernel(x, sw1, sb1, sw2, sb2, rw1, rb1, rw2, rb2, gw, gb):
    grid = (N // BN,)
    full = lambda *s: pl.BlockSpec(s, lambda i: (0,) * len(s))
    return pl.pallas_call(
        _moe_block,
        grid=grid,
        in_specs=[
            pl.BlockSpec((BN, D), lambda i: (i, 0)),
            full(D, 2 * FF), full(2 * FF), full(FF, D), full(D),
            full(2, D, 2 * FF), full(2, 2 * FF), full(2, FF, D), full(2, D),
            full(D, 2), full(2),
        ],
        out_specs=pl.BlockSpec((BN, D), lambda i: (i, 0)),
        out_shape=jax.ShapeDtypeStruct((N, D), jnp.float32),
    )(x, sw1, sb1, sw2, sb2, rw1, rb1, rw2, rb2, gw, gb)
